# Initial kernel scaffold; baseline (speedup 1.0000x reference)
import jax
import jax.numpy as jnp
from jax.experimental import pallas as pl


def kernel(x, edge_index, W1, b1, W2, b2):
    raise NotImplementedError("write your pallas kernel here")


# speedup vs baseline: 14.5304x; 14.5304x over previous
import functools

import jax
import jax.numpy as jnp
from jax import lax
from jax.experimental import pallas as pl
from jax.experimental.pallas import tpu as pltpu
from jax.experimental.pallas import tpu_sc as plsc

N = 10000
E = 320000
IN_DIM = 128
HID_DIM = 256
OUT_DIM = 128

NC = 2
NS = 16
NW = NC * NS

K = 128
CH = 79
EPT = CH * K
EPAD = NW * EPT
NPAD = 10240
DUMMY = NPAD - 1
RPT = NPAD // NS

BLK = 1000
GRID = N // BLK

_mesh = plsc.VectorSubcoreMesh(
    core_axis_name="c", subcore_axis_name="s", num_cores=NC, num_subcores=NS
)


def _deg_body(dst_hbm, ones_hbm, out_hbm, idx_v, ones_v, acc_sh):
    c = lax.axis_index("c")
    s = lax.axis_index("s")
    wid = c * NS + s
    pltpu.sync_copy(dst_hbm.at[wid], idx_v)
    pltpu.sync_copy(ones_hbm, ones_v)

    def zero_step(j, carry):
        pltpu.sync_copy(ones_v.at[pl.ds(0, 128)], acc_sh.at[pl.ds(s * RPT + j * 128, 128)])
        return carry

    lax.fori_loop(0, RPT // 128, zero_step, 0)
    plsc.subcore_barrier()

    def step(j, carry):
        pltpu.sync_copy(ones_v.at[pl.ds(128, K)], acc_sh.at[idx_v.at[j]], add=True)
        return carry

    lax.fori_loop(0, CH, step, 0)
    plsc.subcore_barrier()
    pltpu.sync_copy(acc_sh.at[pl.ds(s * RPT, RPT)], out_hbm.at[c, pl.ds(s * RPT, RPT)])


@functools.partial(jax.jit)
def _deg_call(dst_p, ones16):
    return pl.kernel(
        _deg_body,
        out_type=jax.ShapeDtypeStruct((NC, NPAD, 16), jnp.float32),
        mesh=_mesh,
        scratch_types=[
            pltpu.VMEM((CH, K), jnp.int32),
            pltpu.VMEM((128 + K, 16), jnp.float32),
            pltpu.VMEM_SHARED((NPAD, 16), jnp.float32),
        ],
    )(dst_p, ones16)


def _agg_body(xs_hbm, src_hbm, dst_hbm, zeros_hbm, out_hbm,
              sidx_v, didx_v, rows_v, acc_sh):
    c = lax.axis_index("c")
    s = lax.axis_index("s")
    wid = c * NS + s
    pltpu.sync_copy(src_hbm.at[wid], sidx_v)
    pltpu.sync_copy(dst_hbm.at[wid], didx_v)
    pltpu.sync_copy(zeros_hbm, rows_v)

    def zero_step(j, carry):
        pltpu.sync_copy(rows_v, acc_sh.at[pl.ds(s * RPT + j * 128, 128)])
        return carry

    lax.fori_loop(0, RPT // 128, zero_step, 0)
    plsc.subcore_barrier()

    def step(j, carry):
        pltpu.sync_copy(xs_hbm.at[sidx_v.at[j]], rows_v)
        pltpu.sync_copy(rows_v, acc_sh.at[didx_v.at[j]], add=True)
        return carry

    lax.fori_loop(0, CH, step, 0)
    plsc.subcore_barrier()
    pltpu.sync_copy(acc_sh.at[pl.ds(s * RPT, RPT)], out_hbm.at[c, pl.ds(s * RPT, RPT)])


@functools.partial(jax.jit)
def _agg_call(xs, src_p, dst_p, zeros128):
    return pl.kernel(
        _agg_body,
        out_type=jax.ShapeDtypeStruct((NC, NPAD, IN_DIM), jnp.float32),
        mesh=_mesh,
        scratch_types=[
            pltpu.VMEM((CH, K), jnp.int32),
            pltpu.VMEM((CH, K), jnp.int32),
            pltpu.VMEM((K, IN_DIM), jnp.float32),
            pltpu.VMEM_SHARED((NPAD, IN_DIM), jnp.float32),
        ],
    )(xs, src_p, dst_p, zeros128)


def _dinv_deg(dh_ref):
    deg = dh_ref[0, :, 0:1] + dh_ref[1, :, 0:1] + 1.0
    return lax.rsqrt(deg), deg


def _prep_body(x_ref, dh_ref, xs_ref):
    dinv, _ = _dinv_deg(dh_ref)
    xs_ref[...] = x_ref[...] * dinv


@jax.jit
def _prep_call(x, degh):
    return pl.pallas_call(
        _prep_body,
        grid=(GRID,),
        in_specs=[
            pl.BlockSpec((BLK, IN_DIM), lambda i: (i, 0)),
            pl.BlockSpec((NC, BLK, 16), lambda i: (0, i, 0)),
        ],
        out_specs=pl.BlockSpec((BLK, IN_DIM), lambda i: (i, 0)),
        out_shape=jax.ShapeDtypeStruct((N, IN_DIM), jnp.float32),
    )(x, degh)


def _mid_body(agg_ref, x_ref, dh_ref, w1_ref, b1_ref, w2_ref, ys_ref, y2_ref):
    dinv, deg = _dinv_deg(dh_ref)
    a = agg_ref[0] + agg_ref[1]
    z = a * dinv + x_ref[...] / deg
    h = jnp.dot(z, w1_ref[...], preferred_element_type=jnp.float32) + b1_ref[...]
    h = jnp.maximum(h, 0.0)
    y2 = jnp.dot(h, w2_ref[...], preferred_element_type=jnp.float32)
    y2_ref[...] = y2
    ys_ref[...] = y2 * dinv


@jax.jit
def _mid_call(agg1, x, degh, W1, b1r, W2):
    return pl.pallas_call(
        _mid_body,
        grid=(GRID,),
        in_specs=[
            pl.BlockSpec((NC, BLK, IN_DIM), lambda i: (0, i, 0)),
            pl.BlockSpec((BLK, IN_DIM), lambda i: (i, 0)),
            pl.BlockSpec((NC, BLK, 16), lambda i: (0, i, 0)),
            pl.BlockSpec((IN_DIM, HID_DIM), lambda i: (0, 0)),
            pl.BlockSpec((1, HID_DIM), lambda i: (0, 0)),
            pl.BlockSpec((HID_DIM, OUT_DIM), lambda i: (0, 0)),
        ],
        out_specs=[
            pl.BlockSpec((BLK, OUT_DIM), lambda i: (i, 0)),
            pl.BlockSpec((BLK, OUT_DIM), lambda i: (i, 0)),
        ],
        out_shape=[
            jax.ShapeDtypeStruct((N, OUT_DIM), jnp.float32),
            jax.ShapeDtypeStruct((N, OUT_DIM), jnp.float32),
        ],
    )(agg1, x, degh, W1, b1r, W2)


def _final_body(agg_ref, y2_ref, dh_ref, b2_ref, out_ref):
    dinv, deg = _dinv_deg(dh_ref)
    a = agg_ref[0] + agg_ref[1]
    out_ref[...] = a * dinv + y2_ref[...] / deg + b2_ref[...]


@jax.jit
def _final_call(agg2, y2, degh, b2r):
    return pl.pallas_call(
        _final_body,
        grid=(GRID,),
        in_specs=[
            pl.BlockSpec((NC, BLK, OUT_DIM), lambda i: (0, i, 0)),
            pl.BlockSpec((BLK, OUT_DIM), lambda i: (i, 0)),
            pl.BlockSpec((NC, BLK, 16), lambda i: (0, i, 0)),
            pl.BlockSpec((1, OUT_DIM), lambda i: (0, 0)),
        ],
        out_specs=pl.BlockSpec((BLK, OUT_DIM), lambda i: (i, 0)),
        out_shape=jax.ShapeDtypeStruct((N, OUT_DIM), jnp.float32),
    )(agg2, y2, degh, b2r)


def kernel(x, edge_index, W1, b1, W2, b2):
    src = edge_index[0].astype(jnp.int32)
    dst = edge_index[1].astype(jnp.int32)
    pad = EPAD - E
    src_p = jnp.concatenate([src, jnp.zeros((pad,), jnp.int32)]).reshape(NW, CH, K)
    dst_p = jnp.concatenate([dst, jnp.full((pad,), DUMMY, jnp.int32)]).reshape(NW, CH, K)

    ones16 = jnp.zeros((128 + K, 16), jnp.float32).at[128:, 0].set(1.0)
    zeros128 = jnp.zeros((128, IN_DIM), jnp.float32)
    b1r = b1.reshape(1, HID_DIM)
    b2r = b2.reshape(1, OUT_DIM)

    degh = _deg_call(dst_p, ones16)
    xs = _prep_call(x, degh)
    agg1 = _agg_call(xs, src_p, dst_p, zeros128)
    ys, y2 = _mid_call(agg1, x, degh, W1, b1r, W2)
    agg2 = _agg_call(ys, src_p, dst_p, zeros128)
    out = _final_call(agg2, y2, degh, b2r)
    return out
